# Initial kernel scaffold; baseline (speedup 1.0000x reference)
import jax
import jax.numpy as jnp
from jax.experimental import pallas as pl


def kernel(idx_cat0, idx_cat1, idx_cat2, idx_cat3, emb_cat0, emb_cat1, emb_cat2, emb_cat3):
    raise NotImplementedError("write your pallas kernel here")


# speedup vs baseline: 7.9254x; 7.9254x over previous
import functools

import jax
import jax.numpy as jnp
from jax import lax
from jax.experimental import pallas as pl
from jax.experimental.pallas import tpu as pltpu
from jax.experimental.pallas import tpu_sc as plsc

VOCAB = 100000
DIM = 32
B = 4096
L = 50
N_FIELDS = 4

_TOTAL = B * L
_G = 128
_GROUPS = _TOTAL // _G
_NW = 32
_GPW = _GROUPS // _NW
_GPC = 10
_NCHUNK = _GPW // _GPC
_CHUNK = _GPC * _G


def _make_kernel():
  mesh = plsc.VectorSubcoreMesh(core_axis_name="c", subcore_axis_name="s")

  @functools.partial(
      pl.kernel,
      mesh=mesh,
      compiler_params=pltpu.CompilerParams(use_tc_tiling_on_sc=False),
      out_type=jax.ShapeDtypeStruct((_TOTAL, N_FIELDS * DIM), jnp.float32),
      scratch_types=[
          pltpu.VMEM((_GPW, _G), jnp.int32),
          pltpu.VMEM((_CHUNK, DIM), jnp.float32),
          pltpu.SemaphoreType.DMA,
      ],
  )
  def k(idx0, idx1, idx2, idx3, t0, t1, t2, t3, out, idx_v, rows_v, sem):
    wid = lax.axis_index("s") * 2 + lax.axis_index("c")
    base = wid * (_GPW * _G)
    for f, (idx_hbm, tab) in enumerate(
        ((idx0, t0), (idx1, t1), (idx2, t2), (idx3, t3))):
      pltpu.sync_copy(idx_hbm.at[wid], idx_v)

      def chunk_body(c, carry, tab=tab, f=f):
        copies = []
        for g in range(_GPC):
          copies.append(pltpu.async_copy(
              tab.at[idx_v.at[c * _GPC + g]],
              rows_v.at[pl.ds(g * _G, _G)],
              sem))
        for cp in copies:
          cp.wait()
        row0 = pl.multiple_of(base + c * _CHUNK, 8)
        pltpu.sync_copy(
            rows_v,
            out.at[pl.ds(row0, _CHUNK), pl.ds(f * DIM, DIM)])
        return carry

      lax.fori_loop(0, _NCHUNK, chunk_body, 0)

  return k


_sc_kernel = _make_kernel()


def kernel(idx_cat0, idx_cat1, idx_cat2, idx_cat3,
           emb_cat0, emb_cat1, emb_cat2, emb_cat3):
  idxs = [i.astype(jnp.int32).reshape(_NW, _GPW, _G)
          for i in (idx_cat0, idx_cat1, idx_cat2, idx_cat3)]
  out = _sc_kernel(idxs[0], idxs[1], idxs[2], idxs[3],
                   emb_cat0, emb_cat1, emb_cat2, emb_cat3)
  return out.reshape(B, L, N_FIELDS * DIM)
